# Initial kernel scaffold; baseline (speedup 1.0000x reference)
import jax
import jax.numpy as jnp
from jax.experimental import pallas as pl

_N = 10000
_HEADS = 4
_HID = 64
_CLASSES = 40


def _gat_conv(x, edge_index, W, att_src, att_dst, bias, heads, out_ch, num_nodes):
    src = edge_index[0]
    dst = edge_index[1]
    loop = jnp.arange(num_nodes, dtype=src.dtype)
    src = jnp.concatenate([src, loop])
    dst = jnp.concatenate([dst, loop])
    h = (x @ W).reshape(num_nodes, heads, out_ch)
    a_src = jnp.sum(h * att_src[None, :, :], axis=-1)
    a_dst = jnp.sum(h * att_dst[None, :, :], axis=-1)
    e = a_src[src] + a_dst[dst]
    e = jax.nn.leaky_relu(e, negative_slope=0.2)
    e_max = jax.ops.segment_max(e, dst, num_segments=num_nodes)
    e_max = jax.lax.stop_gradient(jnp.where(jnp.isfinite(e_max), e_max, 0.0))
    e_exp = jnp.exp(e - e_max[dst])
    denom = jax.ops.segment_sum(e_exp, dst, num_segments=num_nodes)
    alpha = e_exp / (denom[dst] + 1e-16)
    msg = h[src] * alpha[:, :, None]
    out = jax.ops.segment_sum(msg, dst, num_segments=num_nodes)
    out = out.reshape(num_nodes, heads * out_ch)
    return out + bias


def _logsoftmax_kernel(x_ref, o_ref):
    v = x_ref[...]
    m = jnp.max(v, axis=1, keepdims=True)
    ex = jnp.exp(v - m)
    lse = jnp.log(jnp.sum(ex, axis=1, keepdims=True))
    o_ref[...] = v - m - lse


def kernel(x, edge_index, W1, att_src1, att_dst1, b1, W2, att_src2, att_dst2, b2):
    h = _gat_conv(x, edge_index, W1, att_src1, att_dst1, b1, _HEADS, _HID, _N)
    h = jax.nn.relu(h)
    out = _gat_conv(h, edge_index, W2, att_src2, att_dst2, b2, 1, _CLASSES, _N)
    return pl.pallas_call(
        _logsoftmax_kernel,
        out_shape=jax.ShapeDtypeStruct((_N, _CLASSES), jnp.float32),
        grid=(10,),
        in_specs=[pl.BlockSpec((_N // 10, _CLASSES), lambda i: (i, 0))],
        out_specs=pl.BlockSpec((_N // 10, _CLASSES), lambda i: (i, 0)),
    )(out)


# speedup vs baseline: 15.9114x; 15.9114x over previous
import jax
import jax.numpy as jnp
from jax import lax
from jax.experimental import pallas as pl
from jax.experimental.pallas import tpu as pltpu
from jax.experimental.pallas import tpu_sc as plsc

N = 10000
F_IN = 128
HID = 64
HEADS = 4
CLASSES = 40

NP = 10240
ROWB = 512
NBLK = NP // ROWB
E_RAW = 320000
E_ALL = E_RAW + N
SLAB = 256
GRP = 16
E2 = 335872
NC, NS, L = 2, 16, 16

C1 = 128
C2 = 48
RW = 128
R2W = 128
RPT = NP // NS

_f32 = jnp.float32
_i32 = jnp.int32

_SC_PARAMS = pltpu.CompilerParams(needs_layout_passes=False)


def _mesh():
    return plsc.VectorSubcoreMesh(core_axis_name="c", subcore_axis_name="s",
                                  num_cores=NC, num_subcores=NS)


def _tc_a_body(x_ref, w1_ref, amat_ref, h_ref, tab_ref, mx_ref):
    h = jnp.dot(x_ref[...], w1_ref[...], preferred_element_type=_f32)
    h_ref[0] = h[:, :C1]
    h_ref[1] = h[:, C1:]
    t = jnp.dot(h, amat_ref[...], preferred_element_type=_f32)
    tab_ref[0] = t[:, :4]
    tab_ref[1] = t[:, 4:]
    b = pl.program_id(0)

    @pl.when(b == 0)
    def _():
        mx_ref[...] = jnp.full((8, 128), -jnp.inf, _f32)

    mx_ref[...] = jnp.maximum(mx_ref[...],
                              jnp.broadcast_to(jnp.max(t, axis=0)[:, None], (8, 128)))


def _tc_a(xp, w1, amat):
    return pl.pallas_call(
        _tc_a_body,
        out_shape=(
            jax.ShapeDtypeStruct((2, NP, C1), _f32),
            jax.ShapeDtypeStruct((2, NP, 4), _f32),
            jax.ShapeDtypeStruct((8, 128), _f32),
        ),
        grid=(NBLK,),
        in_specs=[
            pl.BlockSpec((ROWB, F_IN), lambda b: (b, 0)),
            pl.BlockSpec((F_IN, 2 * C1), lambda b: (0, 0)),
            pl.BlockSpec((2 * C1, 8), lambda b: (0, 0)),
        ],
        out_specs=(
            pl.BlockSpec((2, ROWB, C1), lambda b: (0, b, 0)),
            pl.BlockSpec((2, ROWB, 4), lambda b: (0, b, 0)),
            pl.BlockSpec((8, 128), lambda b: (0, 0)),
        ),
    )(xp, w1, amat)


def _zero_msg(msg, w):
    zv = jnp.zeros((L,), _f32)

    def _zrow(r, _):
        for k in range(w // L):
            msg[r, pl.ds(k * L, L)] = zv
        return 0
    lax.fori_loop(0, GRP, _zrow, 0)


def _zero_acc(msg, acc, row0, w):
    _zero_msg(msg, w)

    def _zcp(i, _):
        pltpu.sync_copy(msg, acc.at[pl.ds(row0 + i * GRP, GRP)])
        return 0
    lax.fori_loop(0, RPT // GRP, _zcp, 0)


def _drain_acc(msg, acc, out_hbm, c, row0):
    def _ocp(i, _):
        pltpu.sync_copy(acc.at[pl.ds(row0 + i * GRP, GRP)], msg)
        pltpu.sync_copy(msg, out_hbm.at[pl.ds(c * NP + row0 + i * GRP, GRP)])
        return 0
    lax.fori_loop(0, RPT // GRP, _ocp, 0)


def _sc_l1_body(esrc, edst, h_all, tab_all, t_arr, out01, out23, outd,
                tab_v, es_v, ed_v, gidx, didx, gbuf, msg, tvec, acc, sem):
    c = lax.axis_index("c")
    s = lax.axis_index("s")
    row0 = s * RPT

    pltpu.sync_copy(t_arr, tvec)
    tb = tvec[...]
    iota = lax.iota(_i32, L)

    edges_per_tile = E2 // (NC * NS)
    tile_e0 = c * (E2 // NC) + s * edges_per_tile

    def _msg_sweep(plane):
        pltpu.sync_copy(tab_all.at[pl.ds(plane * (NP * 4), NP * 4)], tab_v)
        cbase = jnp.full((L,), plane * NP, _i32)

        def _slab(slab_i, _):
            e0 = tile_e0 + slab_i * SLAB
            pltpu.sync_copy(esrc.at[pl.ds(e0, SLAB)], es_v)
            pltpu.sync_copy(edst.at[pl.ds(e0, SLAB)], ed_v)

            def _group(g, _):
                svs = []
                for j in range(GRP // L):
                    sv = es_v[pl.ds(g * GRP + j * L, L)]
                    dv = ed_v[pl.ds(g * GRP + j * L, L)]
                    svs.append(sv)
                    gidx[pl.ds(j * L, L)] = sv + cbase
                    didx[pl.ds(j * L, L)] = dv
                pltpu.async_copy(h_all.at[gidx], gbuf, sem).wait()
                for j in range(GRP // L):
                    sv = svs[j]
                    sb = sv * 4
                    dv = ed_v[pl.ds(g * GRP + j * L, L)]
                    db = dv * 4
                    pv = []
                    for h in range(2):
                        e = (plsc.load_gather(tab_v, [sb + h])
                             + plsc.load_gather(tab_v, [db + (2 + h)]))
                        e = jnp.where(e < 0.0, e * 0.2, e)
                        pv.append(jnp.exp(e - tb))
                    for i in range(L):
                        row = j * L + i
                        p0 = jnp.full((L,), pv[0][i], _f32)
                        p1 = jnp.full((L,), pv[1][i], _f32)
                        for k in range(4):
                            msg[row, pl.ds(k * L, L)] = gbuf[row, pl.ds(k * L, L)] * p0
                        for k in range(4, 8):
                            msg[row, pl.ds(k * L, L)] = gbuf[row, pl.ds(k * L, L)] * p1
                pltpu.sync_copy(msg, acc.at[didx], add=True)
                return 0
            lax.fori_loop(0, SLAB // GRP, _group, 0)
            return 0
        lax.fori_loop(0, edges_per_tile // SLAB, _slab, 0)

    def _den_sweep(plane):
        pltpu.sync_copy(tab_all.at[pl.ds(plane * (NP * 4), NP * 4)], tab_v)

        def _slab(slab_i, _):
            e0 = tile_e0 + slab_i * SLAB
            pltpu.sync_copy(esrc.at[pl.ds(e0, SLAB)], es_v)
            pltpu.sync_copy(edst.at[pl.ds(e0, SLAB)], ed_v)

            def _group(g, _):
                for j in range(GRP // L):
                    sv = es_v[pl.ds(g * GRP + j * L, L)]
                    dv = ed_v[pl.ds(g * GRP + j * L, L)]
                    didx[pl.ds(j * L, L)] = dv
                    sb, db = sv * 4, dv * 4
                    for h in range(2):
                        e = (plsc.load_gather(tab_v, [sb + h])
                             + plsc.load_gather(tab_v, [db + (2 + h)]))
                        e = jnp.where(e < 0.0, e * 0.2, e)
                        p = jnp.exp(e - tb)
                        plsc.store_scatter(
                            msg, [iota + (j * L),
                                  jnp.full((L,), 2 * plane + h, _i32)], p)
                pltpu.sync_copy(msg, acc.at[didx], add=True)
                return 0
            lax.fori_loop(0, SLAB // GRP, _group, 0)
            return 0
        lax.fori_loop(0, edges_per_tile // SLAB, _slab, 0)

    _zero_acc(msg, acc, row0, RW)
    plsc.subcore_barrier()
    _msg_sweep(0)
    plsc.subcore_barrier()
    _drain_acc(msg, acc, out01, c, row0)
    _zero_acc(msg, acc, row0, RW)
    plsc.subcore_barrier()
    _msg_sweep(1)
    plsc.subcore_barrier()
    _drain_acc(msg, acc, out23, c, row0)
    _zero_acc(msg, acc, row0, RW)
    plsc.subcore_barrier()
    _den_sweep(0)
    _zero_msg(msg, RW)
    _den_sweep(1)
    plsc.subcore_barrier()
    _drain_acc(msg, acc, outd, c, row0)


def _sc_l1(esrc, edst, h_all, tab_all, t_arr):
    f = pl.kernel(
        _sc_l1_body,
        out_type=(jax.ShapeDtypeStruct((2 * NP, RW), _f32),
                  jax.ShapeDtypeStruct((2 * NP, RW), _f32),
                  jax.ShapeDtypeStruct((2 * NP, RW), _f32)),
        mesh=_mesh(),
        compiler_params=_SC_PARAMS,
        scratch_types=[
            pltpu.VMEM((NP * 4,), _f32),
            pltpu.VMEM((SLAB,), _i32),
            pltpu.VMEM((SLAB,), _i32),
            pltpu.VMEM((GRP,), _i32),
            pltpu.VMEM((GRP,), _i32),
            pltpu.VMEM((GRP, RW), _f32),
            pltpu.VMEM((GRP, RW), _f32),
            pltpu.VMEM((L,), _f32),
            pltpu.VMEM_SHARED((NP, RW), _f32),
            pltpu.SemaphoreType.DMA,
        ],
    )
    return f(esrc, edst, h_all, tab_all, t_arr)


def _tc_b_body(m01_ref, m23_ref, den_ref, b1_ref, w2_ref, amat2_ref,
               h2_ref, tab2_ref, mx2_ref):
    a0 = m01_ref[0] + m01_ref[1]
    a1 = m23_ref[0] + m23_ref[1]
    den = den_ref[0] + den_ref[1]
    eps = jnp.float32(1e-16)
    g = jnp.concatenate([
        a0[:, 0:64] / (den[:, 0:1] + eps),
        a0[:, 64:128] / (den[:, 1:2] + eps),
        a1[:, 0:64] / (den[:, 2:3] + eps),
        a1[:, 64:128] / (den[:, 3:4] + eps),
    ], axis=1) + b1_ref[...]
    g = jnp.maximum(g, 0.0)
    h2 = jnp.dot(g, w2_ref[...], preferred_element_type=_f32)
    h2_ref[...] = h2
    t2 = jnp.dot(h2, amat2_ref[...], preferred_element_type=_f32)
    tab2_ref[...] = t2[:, :2]
    b = pl.program_id(0)

    @pl.when(b == 0)
    def _():
        mx2_ref[...] = jnp.full((8, 128), -jnp.inf, _f32)

    mx2_ref[...] = jnp.maximum(mx2_ref[...],
                               jnp.broadcast_to(jnp.max(t2, axis=0)[:, None], (8, 128)))


def _tc_b(m01, m23, den, b1r, w2p, amat2):
    acc_spec = pl.BlockSpec((2, ROWB, RW), lambda b: (0, b, 0))
    return pl.pallas_call(
        _tc_b_body,
        out_shape=(
            jax.ShapeDtypeStruct((NP, RW), _f32),
            jax.ShapeDtypeStruct((NP, 2), _f32),
            jax.ShapeDtypeStruct((8, 128), _f32),
        ),
        grid=(NBLK,),
        in_specs=[
            acc_spec,
            acc_spec,
            acc_spec,
            pl.BlockSpec((1, 2 * C1), lambda b: (0, 0)),
            pl.BlockSpec((2 * C1, RW), lambda b: (0, 0)),
            pl.BlockSpec((RW, 8), lambda b: (0, 0)),
        ],
        out_specs=(
            pl.BlockSpec((ROWB, RW), lambda b: (b, 0)),
            pl.BlockSpec((ROWB, 2), lambda b: (b, 0)),
            pl.BlockSpec((8, 128), lambda b: (0, 0)),
        ),
    )(m01, m23, den, b1r, w2p, amat2)


def _sc2_body(esrc, edst, h2p, tab2, t_arr, out_hbm,
              tab_v, es_v, ed_v, gidx, didx, gbuf, msg, tvec, acc, sem):
    c = lax.axis_index("c")
    s = lax.axis_index("s")
    row0 = s * RPT
    _zero_acc(msg, acc, row0, R2W)

    pltpu.sync_copy(tab2, tab_v)
    pltpu.sync_copy(t_arr, tvec)
    tb = tvec[...]
    oh48 = jnp.where(lax.iota(_i32, L) == 0, 1.0, 0.0).astype(_f32)

    plsc.subcore_barrier()

    edges_per_tile = E2 // (NC * NS)
    tile_e0 = c * (E2 // NC) + s * edges_per_tile

    def _slab(slab_i, _):
        e0 = tile_e0 + slab_i * SLAB
        pltpu.sync_copy(esrc.at[pl.ds(e0, SLAB)], es_v)
        pltpu.sync_copy(edst.at[pl.ds(e0, SLAB)], ed_v)

        def _group(g, _):
            svs = []
            for j in range(GRP // L):
                sv = es_v[pl.ds(g * GRP + j * L, L)]
                dv = ed_v[pl.ds(g * GRP + j * L, L)]
                svs.append(sv)
                gidx[pl.ds(j * L, L)] = sv
                didx[pl.ds(j * L, L)] = dv
            pltpu.async_copy(h2p.at[gidx], gbuf, sem).wait()
            for j in range(GRP // L):
                sv = svs[j]
                dv = ed_v[pl.ds(g * GRP + j * L, L)]
                e = (plsc.load_gather(tab_v, [sv * 2])
                     + plsc.load_gather(tab_v, [dv * 2 + 1]))
                e = jnp.where(e < 0.0, e * 0.2, e)
                pvec = jnp.exp(e - tb)
                for i in range(L):
                    row = j * L + i
                    p0 = jnp.full((L,), pvec[i], _f32)
                    for k in range(C2 // L):
                        msg[row, pl.ds(k * L, L)] = gbuf[row, pl.ds(k * L, L)] * p0
                    msg[row, pl.ds(C2, L)] = p0 * oh48
            pltpu.sync_copy(msg, acc.at[didx], add=True)
            return 0
        lax.fori_loop(0, SLAB // GRP, _group, 0)
        return 0
    lax.fori_loop(0, edges_per_tile // SLAB, _slab, 0)

    plsc.subcore_barrier()
    _drain_acc(msg, acc, out_hbm, c, row0)


def _sc2(esrc, edst, h2p, tab2, t_arr):
    f = pl.kernel(
        _sc2_body,
        out_type=jax.ShapeDtypeStruct((2 * NP, R2W), _f32),
        mesh=_mesh(),
        compiler_params=_SC_PARAMS,
        scratch_types=[
            pltpu.VMEM((NP * 2,), _f32),
            pltpu.VMEM((SLAB,), _i32),
            pltpu.VMEM((SLAB,), _i32),
            pltpu.VMEM((GRP,), _i32),
            pltpu.VMEM((GRP,), _i32),
            pltpu.VMEM((GRP, RW), _f32),
            pltpu.VMEM((GRP, R2W), _f32),
            pltpu.VMEM((L,), _f32),
            pltpu.VMEM_SHARED((NP, R2W), _f32),
            pltpu.SemaphoreType.DMA,
        ],
    )
    return f(esrc, edst, h2p, tab2, t_arr)


def _tc_c_body(acc_ref, b2_ref, o_ref):
    a = acc_ref[0] + acc_ref[1]
    v = a[:, 0:C2] / (a[:, C2:C2 + 1] + jnp.float32(1e-16)) + b2_ref[...]
    col = lax.broadcasted_iota(_i32, (ROWB, C2), 1)
    v = jnp.where(col < CLASSES, v, -1e30)
    m = jnp.max(v, axis=1, keepdims=True)
    ex = jnp.exp(v - m)
    lse = jnp.log(jnp.sum(ex, axis=1, keepdims=True))
    o_ref[...] = v - m - lse


def _tc_c(acc2, b2r):
    return pl.pallas_call(
        _tc_c_body,
        out_shape=jax.ShapeDtypeStruct((NP, C2), _f32),
        grid=(NBLK,),
        in_specs=[
            pl.BlockSpec((2, ROWB, R2W), lambda b: (0, b, 0)),
            pl.BlockSpec((1, C2), lambda b: (0, 0)),
        ],
        out_specs=pl.BlockSpec((ROWB, C2), lambda b: (b, 0)),
    )(acc2, b2r)


def kernel(x, edge_index, W1, att_src1, att_dst1, b1, W2, att_src2, att_dst2, b2):
    xp = jnp.pad(x, ((0, NP - N), (0, 0)))
    loop = jnp.arange(N, dtype=_i32)
    pad_cnt = E2 - E_ALL
    pad_idx = N + (jnp.arange(pad_cnt, dtype=_i32) % (NP - N))
    esrc = jnp.concatenate([edge_index[0].astype(_i32), loop, pad_idx])
    edst = jnp.concatenate([edge_index[1].astype(_i32), loop, pad_idx])

    amat = jnp.zeros((2 * C1, 8), _f32)
    for c in range(2):
        for hh in range(2):
            h = c * 2 + hh
            amat = amat.at[h * HID:(h + 1) * HID, c * 4 + hh].set(att_src1[h])
            amat = amat.at[h * HID:(h + 1) * HID, c * 4 + 2 + hh].set(att_dst1[h])
    amat2 = jnp.zeros((RW, 8), _f32)
    amat2 = amat2.at[:CLASSES, 0].set(att_src2[0])
    amat2 = amat2.at[:CLASSES, 1].set(att_dst2[0])
    w2p = jnp.pad(W2, ((0, 0), (0, RW - CLASSES)))
    b1r = b1.reshape(1, 2 * C1)
    b2r = jnp.pad(b2, (0, C2 - CLASSES)).reshape(1, C2)

    h_all, tab, mx = _tc_a(xp, W1, amat)
    t1 = jnp.maximum(jnp.max(mx[jnp.array([0, 1, 4, 5]), 0])
                     + jnp.max(mx[jnp.array([2, 3, 6, 7]), 0]), 0.0)
    t1v = jnp.full((L,), t1, _f32)
    h_flat = h_all.reshape(2 * NP, C1)
    tab_flat = tab.reshape(-1)
    m01, m23, den = _sc_l1(esrc, edst, h_flat, tab_flat, t1v)
    m01 = m01.reshape(2, NP, RW)
    m23 = m23.reshape(2, NP, RW)
    den = den.reshape(2, NP, RW)

    h2p, tab2, mx2 = _tc_b(m01, m23, den, b1r, w2p, amat2)
    t2 = jnp.maximum(mx2[0, 0] + mx2[1, 0], 0.0)
    t2v = jnp.full((L,), t2, _f32)
    acc2 = _sc2(esrc, edst, h2p, tab2.reshape(-1), t2v).reshape(2, NP, R2W)

    out = _tc_c(acc2, b2r)
    return out[:N, :CLASSES]


# speedup vs baseline: 17.6262x; 1.1078x over previous
import jax
import jax.numpy as jnp
from jax import lax
from jax.experimental import pallas as pl
from jax.experimental.pallas import tpu as pltpu
from jax.experimental.pallas import tpu_sc as plsc

N = 10000
F_IN = 128
HID = 64
HEADS = 4
CLASSES = 40

NP = 10240
ROWB = 512
NBLK = NP // ROWB
E_RAW = 320000
E_ALL = E_RAW + N
SLAB = 256
GRP = 16
E2 = 335872
NC, NS, L = 2, 16, 16

C1 = 128
C2 = 48
RW = 128
R2W = 128
RPT = NP // NS

_f32 = jnp.float32
_i32 = jnp.int32

_SC_PARAMS = pltpu.CompilerParams(needs_layout_passes=False)


def _mesh():
    return plsc.VectorSubcoreMesh(core_axis_name="c", subcore_axis_name="s",
                                  num_cores=NC, num_subcores=NS)


def _tc_a_body(x_ref, w1_ref, amat_ref, h_ref, tab_ref, mx_ref):
    h = jnp.dot(x_ref[...], w1_ref[...], preferred_element_type=_f32)
    h_ref[0] = h[:, :C1]
    h_ref[1] = h[:, C1:]
    t = jnp.dot(h, amat_ref[...], preferred_element_type=_f32)
    tab_ref[0] = t[:, :4]
    tab_ref[1] = t[:, 4:]
    b = pl.program_id(0)

    @pl.when(b == 0)
    def _():
        mx_ref[...] = jnp.full((8, 128), -jnp.inf, _f32)

    mx_ref[...] = jnp.maximum(mx_ref[...],
                              jnp.broadcast_to(jnp.max(t, axis=0)[:, None], (8, 128)))


def _tc_a(xp, w1, amat):
    return pl.pallas_call(
        _tc_a_body,
        out_shape=(
            jax.ShapeDtypeStruct((2, NP, C1), _f32),
            jax.ShapeDtypeStruct((2, NP, 4), _f32),
            jax.ShapeDtypeStruct((8, 128), _f32),
        ),
        grid=(NBLK,),
        in_specs=[
            pl.BlockSpec((ROWB, F_IN), lambda b: (b, 0)),
            pl.BlockSpec((F_IN, 2 * C1), lambda b: (0, 0)),
            pl.BlockSpec((2 * C1, 8), lambda b: (0, 0)),
        ],
        out_specs=(
            pl.BlockSpec((2, ROWB, C1), lambda b: (0, b, 0)),
            pl.BlockSpec((2, ROWB, 4), lambda b: (0, b, 0)),
            pl.BlockSpec((8, 128), lambda b: (0, 0)),
        ),
    )(xp, w1, amat)


def _zero_msg(msg, w):
    zv = jnp.zeros((L,), _f32)

    def _zrow(r, _):
        for k in range(w // L):
            msg[r, pl.ds(k * L, L)] = zv
        return 0
    lax.fori_loop(0, GRP, _zrow, 0)


def _zero_acc(msg, acc, row0, w):
    _zero_msg(msg, w)

    def _zcp(i, _):
        pltpu.sync_copy(msg, acc.at[pl.ds(row0 + i * GRP, GRP)])
        return 0
    lax.fori_loop(0, RPT // GRP, _zcp, 0)


def _drain_acc(msg, acc, out_hbm, c, row0):
    def _ocp(i, _):
        pltpu.sync_copy(acc.at[pl.ds(row0 + i * GRP, GRP)], msg)
        pltpu.sync_copy(msg, out_hbm.at[pl.ds(c * NP + row0 + i * GRP, GRP)])
        return 0
    lax.fori_loop(0, RPT // GRP, _ocp, 0)


def _sc_l1_body(esrc, edst, h_all, tab_all, t_arr, out01, out23, outd,
                tab_v, es_v, ed_v, gidx, didx, gbuf, msg, tvec, acc, sem, ssem):
    c = lax.axis_index("c")
    s = lax.axis_index("s")
    row0 = s * RPT

    pltpu.sync_copy(t_arr, tvec)
    tb = tvec[...]
    iota = lax.iota(_i32, L)

    edges_per_tile = E2 // (NC * NS)
    tile_e0 = c * (E2 // NC) + s * edges_per_tile

    ngroups = SLAB // GRP

    def _msg_sweep(plane):
        pltpu.sync_copy(tab_all.at[pl.ds(plane * (NP * 4), NP * 4)], tab_v)
        cbase = jnp.full((L,), plane * NP, _i32)

        def _slab(slab_i, _):
            e0 = tile_e0 + slab_i * SLAB
            pltpu.sync_copy(esrc.at[pl.ds(e0, SLAB)], es_v)
            pltpu.sync_copy(edst.at[pl.ds(e0, SLAB)], ed_v)
            for j in range(GRP // L):
                gidx[pl.ds(j * L, L)] = es_v[pl.ds(j * L, L)] + cbase
            pltpu.async_copy(h_all.at[gidx], gbuf, sem)

            def _group(g, _):
                pltpu.make_async_copy(h_all.at[gidx], gbuf, sem).wait()

                @pl.when(g > 0)
                def _():
                    pltpu.make_async_copy(msg, acc.at[didx], ssem).wait()

                for j in range(GRP // L):
                    dv = ed_v[pl.ds(g * GRP + j * L, L)]
                    didx[pl.ds(j * L, L)] = dv
                for j in range(GRP // L):
                    sv = es_v[pl.ds(g * GRP + j * L, L)]
                    sb = sv * 4
                    dv = ed_v[pl.ds(g * GRP + j * L, L)]
                    db = dv * 4
                    pv = []
                    for h in range(2):
                        e = (plsc.load_gather(tab_v, [sb + h])
                             + plsc.load_gather(tab_v, [db + (2 + h)]))
                        e = jnp.where(e < 0.0, e * 0.2, e)
                        pv.append(jnp.exp(e - tb))
                    for i in range(L):
                        row = j * L + i
                        p0 = jnp.full((L,), pv[0][i], _f32)
                        p1 = jnp.full((L,), pv[1][i], _f32)
                        for k in range(4):
                            msg[row, pl.ds(k * L, L)] = gbuf[row, pl.ds(k * L, L)] * p0
                        for k in range(4, 8):
                            msg[row, pl.ds(k * L, L)] = gbuf[row, pl.ds(k * L, L)] * p1
                pltpu.async_copy(msg, acc.at[didx], ssem, add=True)

                @pl.when(g < ngroups - 1)
                def _():
                    for j in range(GRP // L):
                        svn = es_v[pl.ds((g + 1) * GRP + j * L, L)]
                        gidx[pl.ds(j * L, L)] = svn + cbase
                    pltpu.async_copy(h_all.at[gidx], gbuf, sem)
                return 0
            lax.fori_loop(0, ngroups, _group, 0)
            pltpu.make_async_copy(msg, acc.at[didx], ssem).wait()
            return 0
        lax.fori_loop(0, edges_per_tile // SLAB, _slab, 0)

    def _den_sweep(plane):
        pltpu.sync_copy(tab_all.at[pl.ds(plane * (NP * 4), NP * 4)], tab_v)

        def _slab(slab_i, _):
            e0 = tile_e0 + slab_i * SLAB
            pltpu.sync_copy(esrc.at[pl.ds(e0, SLAB)], es_v)
            pltpu.sync_copy(edst.at[pl.ds(e0, SLAB)], ed_v)

            def _group(g, _):
                @pl.when(g > 0)
                def _():
                    pltpu.make_async_copy(msg, acc.at[didx], ssem).wait()

                for j in range(GRP // L):
                    sv = es_v[pl.ds(g * GRP + j * L, L)]
                    dv = ed_v[pl.ds(g * GRP + j * L, L)]
                    didx[pl.ds(j * L, L)] = dv
                    sb, db = sv * 4, dv * 4
                    for h in range(2):
                        e = (plsc.load_gather(tab_v, [sb + h])
                             + plsc.load_gather(tab_v, [db + (2 + h)]))
                        e = jnp.where(e < 0.0, e * 0.2, e)
                        p = jnp.exp(e - tb)
                        plsc.store_scatter(
                            msg, [iota + (j * L),
                                  jnp.full((L,), 2 * plane + h, _i32)], p)
                pltpu.async_copy(msg, acc.at[didx], ssem, add=True)
                return 0
            lax.fori_loop(0, SLAB // GRP, _group, 0)
            pltpu.make_async_copy(msg, acc.at[didx], ssem).wait()
            return 0
        lax.fori_loop(0, edges_per_tile // SLAB, _slab, 0)

    _zero_acc(msg, acc, row0, RW)
    plsc.subcore_barrier()
    _msg_sweep(0)
    plsc.subcore_barrier()
    _drain_acc(msg, acc, out01, c, row0)
    _zero_acc(msg, acc, row0, RW)
    plsc.subcore_barrier()
    _msg_sweep(1)
    plsc.subcore_barrier()
    _drain_acc(msg, acc, out23, c, row0)
    _zero_acc(msg, acc, row0, RW)
    plsc.subcore_barrier()
    _den_sweep(0)
    _zero_msg(msg, RW)
    _den_sweep(1)
    plsc.subcore_barrier()
    _drain_acc(msg, acc, outd, c, row0)


def _sc_l1(esrc, edst, h_all, tab_all, t_arr):
    f = pl.kernel(
        _sc_l1_body,
        out_type=(jax.ShapeDtypeStruct((2 * NP, RW), _f32),
                  jax.ShapeDtypeStruct((2 * NP, RW), _f32),
                  jax.ShapeDtypeStruct((2 * NP, RW), _f32)),
        mesh=_mesh(),
        compiler_params=_SC_PARAMS,
        scratch_types=[
            pltpu.VMEM((NP * 4,), _f32),
            pltpu.VMEM((SLAB,), _i32),
            pltpu.VMEM((SLAB,), _i32),
            pltpu.VMEM((GRP,), _i32),
            pltpu.VMEM((GRP,), _i32),
            pltpu.VMEM((GRP, RW), _f32),
            pltpu.VMEM((GRP, RW), _f32),
            pltpu.VMEM((L,), _f32),
            pltpu.VMEM_SHARED((NP, RW), _f32),
            pltpu.SemaphoreType.DMA,
            pltpu.SemaphoreType.DMA,
        ],
    )
    return f(esrc, edst, h_all, tab_all, t_arr)


def _tc_b_body(m01_ref, m23_ref, den_ref, b1_ref, w2_ref, amat2_ref,
               h2_ref, tab2_ref, mx2_ref):
    a0 = m01_ref[0] + m01_ref[1]
    a1 = m23_ref[0] + m23_ref[1]
    den = den_ref[0] + den_ref[1]
    eps = jnp.float32(1e-16)
    g = jnp.concatenate([
        a0[:, 0:64] / (den[:, 0:1] + eps),
        a0[:, 64:128] / (den[:, 1:2] + eps),
        a1[:, 0:64] / (den[:, 2:3] + eps),
        a1[:, 64:128] / (den[:, 3:4] + eps),
    ], axis=1) + b1_ref[...]
    g = jnp.maximum(g, 0.0)
    h2 = jnp.dot(g, w2_ref[...], preferred_element_type=_f32)
    h2_ref[...] = h2
    t2 = jnp.dot(h2, amat2_ref[...], preferred_element_type=_f32)
    tab2_ref[...] = t2[:, :2]
    b = pl.program_id(0)

    @pl.when(b == 0)
    def _():
        mx2_ref[...] = jnp.full((8, 128), -jnp.inf, _f32)

    mx2_ref[...] = jnp.maximum(mx2_ref[...],
                               jnp.broadcast_to(jnp.max(t2, axis=0)[:, None], (8, 128)))


def _tc_b(m01, m23, den, b1r, w2p, amat2):
    acc_spec = pl.BlockSpec((2, ROWB, RW), lambda b: (0, b, 0))
    return pl.pallas_call(
        _tc_b_body,
        out_shape=(
            jax.ShapeDtypeStruct((NP, RW), _f32),
            jax.ShapeDtypeStruct((NP, 2), _f32),
            jax.ShapeDtypeStruct((8, 128), _f32),
        ),
        grid=(NBLK,),
        in_specs=[
            acc_spec,
            acc_spec,
            acc_spec,
            pl.BlockSpec((1, 2 * C1), lambda b: (0, 0)),
            pl.BlockSpec((2 * C1, RW), lambda b: (0, 0)),
            pl.BlockSpec((RW, 8), lambda b: (0, 0)),
        ],
        out_specs=(
            pl.BlockSpec((ROWB, RW), lambda b: (b, 0)),
            pl.BlockSpec((ROWB, 2), lambda b: (b, 0)),
            pl.BlockSpec((8, 128), lambda b: (0, 0)),
        ),
    )(m01, m23, den, b1r, w2p, amat2)


def _sc2_body(esrc, edst, h2p, tab2, t_arr, out_hbm,
              tab_v, es_v, ed_v, gidx, didx, gbuf, msg, tvec, acc, sem, ssem):
    c = lax.axis_index("c")
    s = lax.axis_index("s")
    row0 = s * RPT
    _zero_acc(msg, acc, row0, R2W)

    pltpu.sync_copy(tab2, tab_v)
    pltpu.sync_copy(t_arr, tvec)
    tb = tvec[...]
    oh48 = jnp.where(lax.iota(_i32, L) == 0, 1.0, 0.0).astype(_f32)

    plsc.subcore_barrier()

    edges_per_tile = E2 // (NC * NS)
    tile_e0 = c * (E2 // NC) + s * edges_per_tile

    ngroups = SLAB // GRP

    def _slab(slab_i, _):
        e0 = tile_e0 + slab_i * SLAB
        pltpu.sync_copy(esrc.at[pl.ds(e0, SLAB)], es_v)
        pltpu.sync_copy(edst.at[pl.ds(e0, SLAB)], ed_v)
        for j in range(GRP // L):
            gidx[pl.ds(j * L, L)] = es_v[pl.ds(j * L, L)]
        pltpu.async_copy(h2p.at[gidx], gbuf, sem)

        def _group(g, _):
            pltpu.make_async_copy(h2p.at[gidx], gbuf, sem).wait()

            @pl.when(g > 0)
            def _():
                pltpu.make_async_copy(msg, acc.at[didx], ssem).wait()

            for j in range(GRP // L):
                dv = ed_v[pl.ds(g * GRP + j * L, L)]
                didx[pl.ds(j * L, L)] = dv
            for j in range(GRP // L):
                sv = es_v[pl.ds(g * GRP + j * L, L)]
                dv = ed_v[pl.ds(g * GRP + j * L, L)]
                e = (plsc.load_gather(tab_v, [sv * 2])
                     + plsc.load_gather(tab_v, [dv * 2 + 1]))
                e = jnp.where(e < 0.0, e * 0.2, e)
                pvec = jnp.exp(e - tb)
                for i in range(L):
                    row = j * L + i
                    p0 = jnp.full((L,), pvec[i], _f32)
                    for k in range(C2 // L):
                        msg[row, pl.ds(k * L, L)] = gbuf[row, pl.ds(k * L, L)] * p0
                    msg[row, pl.ds(C2, L)] = p0 * oh48
            pltpu.async_copy(msg, acc.at[didx], ssem, add=True)

            @pl.when(g < ngroups - 1)
            def _():
                for j in range(GRP // L):
                    gidx[pl.ds(j * L, L)] = es_v[pl.ds((g + 1) * GRP + j * L, L)]
                pltpu.async_copy(h2p.at[gidx], gbuf, sem)
            return 0
        lax.fori_loop(0, ngroups, _group, 0)
        pltpu.make_async_copy(msg, acc.at[didx], ssem).wait()
        return 0
    lax.fori_loop(0, edges_per_tile // SLAB, _slab, 0)

    plsc.subcore_barrier()
    _drain_acc(msg, acc, out_hbm, c, row0)


def _sc2(esrc, edst, h2p, tab2, t_arr):
    f = pl.kernel(
        _sc2_body,
        out_type=jax.ShapeDtypeStruct((2 * NP, R2W), _f32),
        mesh=_mesh(),
        compiler_params=_SC_PARAMS,
        scratch_types=[
            pltpu.VMEM((NP * 2,), _f32),
            pltpu.VMEM((SLAB,), _i32),
            pltpu.VMEM((SLAB,), _i32),
            pltpu.VMEM((GRP,), _i32),
            pltpu.VMEM((GRP,), _i32),
            pltpu.VMEM((GRP, RW), _f32),
            pltpu.VMEM((GRP, R2W), _f32),
            pltpu.VMEM((L,), _f32),
            pltpu.VMEM_SHARED((NP, R2W), _f32),
            pltpu.SemaphoreType.DMA,
            pltpu.SemaphoreType.DMA,
        ],
    )
    return f(esrc, edst, h2p, tab2, t_arr)


def _tc_c_body(acc_ref, b2_ref, o_ref):
    a = acc_ref[0] + acc_ref[1]
    v = a[:, 0:C2] / (a[:, C2:C2 + 1] + jnp.float32(1e-16)) + b2_ref[...]
    col = lax.broadcasted_iota(_i32, (ROWB, C2), 1)
    v = jnp.where(col < CLASSES, v, -1e30)
    m = jnp.max(v, axis=1, keepdims=True)
    ex = jnp.exp(v - m)
    lse = jnp.log(jnp.sum(ex, axis=1, keepdims=True))
    o_ref[...] = v - m - lse


def _tc_c(acc2, b2r):
    return pl.pallas_call(
        _tc_c_body,
        out_shape=jax.ShapeDtypeStruct((NP, C2), _f32),
        grid=(NBLK,),
        in_specs=[
            pl.BlockSpec((2, ROWB, R2W), lambda b: (0, b, 0)),
            pl.BlockSpec((1, C2), lambda b: (0, 0)),
        ],
        out_specs=pl.BlockSpec((ROWB, C2), lambda b: (b, 0)),
    )(acc2, b2r)


def kernel(x, edge_index, W1, att_src1, att_dst1, b1, W2, att_src2, att_dst2, b2):
    xp = jnp.pad(x, ((0, NP - N), (0, 0)))
    loop = jnp.arange(N, dtype=_i32)
    pad_cnt = E2 - E_ALL
    pad_idx = N + (jnp.arange(pad_cnt, dtype=_i32) % (NP - N))
    esrc = jnp.concatenate([edge_index[0].astype(_i32), loop, pad_idx])
    edst = jnp.concatenate([edge_index[1].astype(_i32), loop, pad_idx])

    amat = jnp.zeros((2 * C1, 8), _f32)
    for c in range(2):
        for hh in range(2):
            h = c * 2 + hh
            amat = amat.at[h * HID:(h + 1) * HID, c * 4 + hh].set(att_src1[h])
            amat = amat.at[h * HID:(h + 1) * HID, c * 4 + 2 + hh].set(att_dst1[h])
    amat2 = jnp.zeros((RW, 8), _f32)
    amat2 = amat2.at[:CLASSES, 0].set(att_src2[0])
    amat2 = amat2.at[:CLASSES, 1].set(att_dst2[0])
    w2p = jnp.pad(W2, ((0, 0), (0, RW - CLASSES)))
    b1r = b1.reshape(1, 2 * C1)
    b2r = jnp.pad(b2, (0, C2 - CLASSES)).reshape(1, C2)

    h_all, tab, mx = _tc_a(xp, W1, amat)
    t1 = jnp.maximum(jnp.max(mx[jnp.array([0, 1, 4, 5]), 0])
                     + jnp.max(mx[jnp.array([2, 3, 6, 7]), 0]), 0.0)
    t1v = jnp.full((L,), t1, _f32)
    h_flat = h_all.reshape(2 * NP, C1)
    tab_flat = tab.reshape(-1)
    m01, m23, den = _sc_l1(esrc, edst, h_flat, tab_flat, t1v)
    m01 = m01.reshape(2, NP, RW)
    m23 = m23.reshape(2, NP, RW)
    den = den.reshape(2, NP, RW)

    h2p, tab2, mx2 = _tc_b(m01, m23, den, b1r, w2p, amat2)
    t2 = jnp.maximum(mx2[0, 0] + mx2[1, 0], 0.0)
    t2v = jnp.full((L,), t2, _f32)
    acc2 = _sc2(esrc, edst, h2p, tab2.reshape(-1), t2v).reshape(2, NP, R2W)

    out = _tc_c(acc2, b2r)
    return out[:N, :CLASSES]
